# baseline (speedup 1.0000x reference)
import functools

import jax
import jax.numpy as jnp
from jax import lax
from jax.experimental import pallas as pl
from jax.experimental.pallas import tpu as pltpu
from jax.experimental.pallas import tpu_sc as plsc

N_VOCAB = 1000000
D_MODEL = 32
BATCH = 4096
HIST = 200

NC = 2
NS = 16
NW = NC * NS
BPW = BATCH // NW

_mesh = plsc.VectorSubcoreMesh(core_axis_name="c", subcore_axis_name="s")


@functools.partial(
    pl.kernel,
    mesh=_mesh,
    out_type=jax.ShapeDtypeStruct((HIST, 4, NW, 8, 128), jnp.float32),
    scratch_types=[
        pltpu.VMEM((HIST, BPW), jnp.int32),
        pltpu.VMEM((2, BPW, D_MODEL), jnp.float32),
        pltpu.VMEM((2, 4, 8, 128), jnp.float32),
        pltpu.SemaphoreType.DMA,
        pltpu.SemaphoreType.DMA,
        pltpu.SemaphoreType.DMA,
        pltpu.SemaphoreType.DMA,
    ],
    compiler_params=pltpu.CompilerParams(
        use_tc_tiling_on_sc=False,
        needs_layout_passes=False,
        disable_bounds_checks=True,
    ),
)
def _embed_gather(table_hbm, idx_hbm, out_hbm, idx_v, stage_v, trans_v,
                  gsem0, gsem1, wsem0, wsem1):
    wid = lax.axis_index("s") * NC + lax.axis_index("c")
    gsems = (gsem0, gsem1)
    wsems = (wsem0, wsem1)
    iota16 = lax.iota(jnp.int32, 16)

    pltpu.sync_copy(idx_hbm.at[wid], idx_v)

    def issue_gather(h, b):
        pltpu.async_copy(
            table_hbm.at[idx_v.at[h]], stage_v.at[b], gsems[b]
        )

    def wait_gather(b):
        pltpu.make_async_copy(
            table_hbm.at[pl.ds(0, BPW)], stage_v.at[b], gsems[b]
        ).wait()

    zero16 = iota16 * 0

    def transpose_block(b):
        @plsc.parallel_loop(0, BPW // 16, unroll=4)
        def _tl(l):
            i_vec = iota16 + 16 * l
            for d in range(D_MODEL):
                vals = plsc.load_gather(stage_v.at[b], [i_vec, zero16 + d])
                trans_v[b, d // 8, d % 8, pl.ds(16 * l, 16)] = vals

    def issue_write(h, b):
        pltpu.async_copy(
            trans_v.at[b], out_hbm.at[h, :, wid], wsems[b]
        )

    def wait_write(b):
        pltpu.make_async_copy(
            trans_v.at[b], out_hbm.at[0, :, wid], wsems[b]
        ).wait()

    issue_gather(0, 0)
    issue_gather(1, 1)
    wait_gather(0)
    transpose_block(0)
    issue_write(0, 0)

    def pair_body(p, carry):
        for b in range(2):
            h = 2 * p + b
            wait_write(b)
            issue_gather(h, b)
            wait_gather(1 - b)
            transpose_block(1 - b)
            issue_write(h - 1, 1 - b)
        return carry

    lax.fori_loop(1, HIST // 2, pair_body, 0)

    wait_gather(1)
    transpose_block(1)
    issue_write(HIST - 1, 1)
    wait_write(0)
    wait_write(1)


def kernel(tok_ids, embedding):
    idx = tok_ids.T.astype(jnp.int32).reshape(HIST, NW, BPW).transpose(1, 0, 2)
    out5 = _embed_gather(embedding, idx)
    return jnp.transpose(out5, (2, 4, 0, 1, 3)).reshape(BATCH, HIST, D_MODEL)


# speedup vs baseline: 1.0395x; 1.0395x over previous
import functools

import jax
import jax.numpy as jnp
from jax import lax
from jax.experimental import pallas as pl
from jax.experimental.pallas import tpu as pltpu
from jax.experimental.pallas import tpu_sc as plsc

N_VOCAB = 1000000
D_MODEL = 32
BATCH = 4096
HIST = 200

NC = 2
NS = 16
NW = NC * NS
BPW = BATCH // NW

_mesh = plsc.VectorSubcoreMesh(core_axis_name="c", subcore_axis_name="s")


FMT_ROWS = 512
FMT_BLKS = FMT_ROWS // 32
FMT_CHUNKS = N_VOCAB // FMT_ROWS
FMT_TAIL = N_VOCAB - FMT_CHUNKS * FMT_ROWS
FMT_ITERS = (FMT_CHUNKS + NW - 1) // NW


@functools.partial(
    pl.kernel,
    mesh=_mesh,
    out_type=jax.ShapeDtypeStruct((N_VOCAB // 32, 8, 128), jnp.float32),
    scratch_types=[
        pltpu.VMEM((2, D_MODEL, FMT_ROWS), jnp.float32),
        pltpu.VMEM((2, FMT_BLKS, 8, 128), jnp.float32),
        pltpu.SemaphoreType.DMA,
        pltpu.SemaphoreType.DMA,
        pltpu.SemaphoreType.DMA,
        pltpu.SemaphoreType.DMA,
    ],
    compiler_params=pltpu.CompilerParams(
        use_tc_tiling_on_sc=True,
        needs_layout_passes=False,
        disable_bounds_checks=True,
    ),
)
def _table_format(embt_hbm, tail_hbm, out_hbm, src_v, dst_v, rs0, rs1, ws0, ws1):
    wid = lax.axis_index("s") * NC + lax.axis_index("c")
    rsems = (rs0, rs1)
    wsems = (ws0, ws1)
    iota16 = lax.iota(jnp.int32, 16)
    zero16 = iota16 * 0
    dvecs = (iota16, iota16 + 16)

    def chunk_id(k):
        return wid + NW * k

    def issue_read(k, b):
        @pl.when(chunk_id(k) < FMT_CHUNKS)
        def _():
            c0 = chunk_id(k) * FMT_ROWS
            pltpu.async_copy(
                embt_hbm.at[:, pl.ds(c0, FMT_ROWS)], src_v.at[b], rsems[b]
            )

    def wait_read(k, b):
        @pl.when(chunk_id(k) < FMT_CHUNKS)
        def _():
            pltpu.make_async_copy(
                embt_hbm.at[:, pl.ds(0, FMT_ROWS)], src_v.at[b], rsems[b]
            ).wait()

    def transpose_chunk(b):
        @plsc.parallel_loop(0, FMT_BLKS, unroll=2)
        def _tp(t):
            for i in range(8):
                for m in range(8):
                    j = 32 * t + 4 * i + m // 2
                    vals = plsc.load_gather(
                        src_v.at[b], [dvecs[m % 2], zero16 + j]
                    )
                    dst_v[b, t, i, pl.ds(16 * m, 16)] = vals

    def issue_write(k, b):
        @pl.when(chunk_id(k) < FMT_CHUNKS)
        def _():
            pltpu.async_copy(
                dst_v.at[b],
                out_hbm.at[pl.ds(chunk_id(k) * FMT_BLKS, FMT_BLKS)],
                wsems[b],
            )

    def wait_write(k, b):
        @pl.when(chunk_id(k) < FMT_CHUNKS)
        def _():
            pltpu.make_async_copy(
                dst_v.at[b], out_hbm.at[pl.ds(0, FMT_BLKS)], wsems[b]
            ).wait()

    issue_read(0, 0)

    def body(k, carry):
        for b in range(2):
            kk = 2 * k + b

            @pl.when(kk < FMT_ITERS)
            def _():
                wait_read(kk, b)
                issue_read(kk + 1, 1 - b)

                @pl.when(kk >= 2)
                def _():
                    wait_write(kk - 2, b)

                @pl.when(chunk_id(kk) < FMT_CHUNKS)
                def _():
                    transpose_chunk(b)

                issue_write(kk, b)
        return carry

    lax.fori_loop(0, (FMT_ITERS + 1) // 2, body, 0)
    wait_write(FMT_ITERS - 2, FMT_ITERS % 2)
    wait_write(FMT_ITERS - 1, 1 - FMT_ITERS % 2)

    @pl.when(wid == 0)
    def _():
        pltpu.sync_copy(tail_hbm, dst_v.at[0, pl.ds(0, FMT_TAIL // 32)])
        pltpu.sync_copy(
            dst_v.at[0, pl.ds(0, FMT_TAIL // 32)],
            out_hbm.at[pl.ds(FMT_CHUNKS * FMT_ROWS // 32, FMT_TAIL // 32)],
        )


@functools.partial(
    pl.kernel,
    mesh=_mesh,
    out_type=jax.ShapeDtypeStruct((HIST, 4, NW, 8, 128), jnp.float32),
    scratch_types=[
        pltpu.VMEM((HIST, BPW), jnp.int32),
        pltpu.VMEM((2, BPW, D_MODEL), jnp.float32),
        pltpu.VMEM((2, 4, 8, 128), jnp.float32),
        pltpu.SemaphoreType.DMA,
        pltpu.SemaphoreType.DMA,
        pltpu.SemaphoreType.DMA,
        pltpu.SemaphoreType.DMA,
    ],
    compiler_params=pltpu.CompilerParams(
        use_tc_tiling_on_sc=False,
        needs_layout_passes=False,
        disable_bounds_checks=True,
    ),
)
def _embed_gather(table_hbm, idx_hbm, out_hbm, idx_v, stage_v, trans_v,
                  gsem0, gsem1, wsem0, wsem1):
    wid = lax.axis_index("s") * NC + lax.axis_index("c")
    gsems = (gsem0, gsem1)
    wsems = (wsem0, wsem1)
    iota16 = lax.iota(jnp.int32, 16)

    pltpu.sync_copy(idx_hbm.at[wid], idx_v)

    def issue_gather(h, b):
        pltpu.async_copy(
            table_hbm.at[idx_v.at[h]], stage_v.at[b], gsems[b]
        )

    def wait_gather(b):
        pltpu.make_async_copy(
            table_hbm.at[pl.ds(0, BPW)], stage_v.at[b], gsems[b]
        ).wait()

    zero16 = iota16 * 0

    def transpose_block(b):
        @plsc.parallel_loop(0, BPW // 16, unroll=2)
        def _tl(l):
            i_vec = iota16 + 16 * l
            for d in range(D_MODEL):
                vals = plsc.load_gather(stage_v.at[b], [i_vec, zero16 + d])
                trans_v[b, d // 8, d % 8, pl.ds(16 * l, 16)] = vals

    def issue_write(h, b):
        pltpu.async_copy(
            trans_v.at[b], out_hbm.at[h, :, wid], wsems[b]
        )

    def wait_write(b):
        pltpu.make_async_copy(
            trans_v.at[b], out_hbm.at[0, :, wid], wsems[b]
        ).wait()

    issue_gather(0, 0)
    issue_gather(1, 1)
    wait_gather(0)
    transpose_block(0)
    issue_write(0, 0)

    def pair_body(p, carry):
        for b in range(2):
            h = 2 * p + b
            wait_write(b)
            issue_gather(h, b)
            wait_gather(1 - b)
            transpose_block(1 - b)
            issue_write(h - 1, 1 - b)
        return carry

    lax.fori_loop(1, HIST // 2, pair_body, 0)

    wait_gather(1)
    transpose_block(1)
    issue_write(HIST - 1, 1)
    wait_write(0)
    wait_write(1)


def kernel(tok_ids, embedding):
    idx = tok_ids.T.astype(jnp.int32).reshape(HIST, NW, BPW).transpose(1, 0, 2)
    tail = embedding[FMT_CHUNKS * FMT_ROWS:, :].reshape(FMT_TAIL // 32, 8, 128)
    table = _table_format(embedding.T, tail).reshape(N_VOCAB, D_MODEL)
    out5 = _embed_gather(table, idx)
    return jnp.transpose(out5, (2, 4, 0, 1, 3)).reshape(BATCH, HIST, D_MODEL)
